# baseline (speedup 1.0000x reference)
import jax
import jax.numpy as jnp
from jax.experimental import pallas as pl

B, N, D_IN, HIDDEN = 4096, 50, 64, 128
M = B * N
TILE = 4096


def _tok_kernel(x_ref, m_ref, w1_ref, w2_ref, b1_ref, b2_ref, o_ref):
    w = w1_ref[...] + w2_ref[...]
    b = b1_ref[...] + b2_ref[...]
    acc = jax.lax.dot_general(
        x_ref[...], w,
        dimension_numbers=(((1,), (0,)), ((), ())),
        preferred_element_type=jnp.float32,
    )
    o_ref[...] = (acc + b) * m_ref[...]


def kernel(x_feats, feats_masks, W1, b1, W2, b2):
    x2 = x_feats.reshape(M, D_IN)
    m2 = feats_masks.reshape(M, 1).astype(jnp.float32)
    b1r = b1.reshape(1, HIDDEN)
    b2r = b2.reshape(1, HIDDEN)

    grid = (M // TILE,)
    out = pl.pallas_call(
        _tok_kernel,
        grid=grid,
        in_specs=[
            pl.BlockSpec((TILE, D_IN), lambda i: (i, 0)),
            pl.BlockSpec((TILE, 1), lambda i: (i, 0)),
            pl.BlockSpec((D_IN, HIDDEN), lambda i: (0, 0)),
            pl.BlockSpec((D_IN, HIDDEN), lambda i: (0, 0)),
            pl.BlockSpec((1, HIDDEN), lambda i: (0, 0)),
            pl.BlockSpec((1, HIDDEN), lambda i: (0, 0)),
        ],
        out_specs=pl.BlockSpec((TILE, HIDDEN), lambda i: (i, 0)),
        out_shape=jax.ShapeDtypeStruct((M, HIDDEN), jnp.float32),
    )(x2, m2, W1, W2, b1r, b2r)
    return out.reshape(B, N, HIDDEN)


# speedup vs baseline: 1.9505x; 1.9505x over previous
import jax
import jax.numpy as jnp
from jax.experimental import pallas as pl

B, N, D_IN, HIDDEN = 4096, 50, 64, 128
NP = 56
BT = 128
R = BT * NP


def _tok_kernel(x_ref, m_ref, e_ref, f_ref, w1_ref, w2_ref, b1_ref, b2_ref, o_ref):
    w = w1_ref[...] + w2_ref[...]
    b = b1_ref[...] + b2_ref[...]
    x3 = jax.lax.pad(x_ref[...], jnp.float32(0),
                     ((0, 0, 0), (0, NP - N, 0), (0, 0, 0)))
    x2 = x3.reshape(R, D_IN)
    acc = jax.lax.dot_general(
        x2, w,
        dimension_numbers=(((1,), (0,)), ((), ())),
        preferred_element_type=jnp.float32,
    )
    g = jax.lax.dot_general(
        e_ref[...], m_ref[...],
        dimension_numbers=(((1,), (0,)), ((), ())),
        preferred_element_type=jnp.float32,
    )
    mask_full = jax.lax.dot_general(
        g * f_ref[...], jnp.ones((N, HIDDEN), jnp.float32),
        dimension_numbers=(((1,), (0,)), ((), ())),
        preferred_element_type=jnp.float32,
    )
    res = (acc + b) * mask_full
    o_ref[...] = res.reshape(BT, NP, HIDDEN)[:, :N, :]


def kernel(x_feats, feats_masks, W1, b1, W2, b2):
    mf = feats_masks.astype(jnp.bfloat16)
    b1r = b1.reshape(1, HIDDEN)
    b2r = b2.reshape(1, HIDDEN)
    r_idx = jnp.arange(R, dtype=jnp.int32)
    e_mat = (r_idx[:, None] // NP == jnp.arange(BT, dtype=jnp.int32)[None, :]
             ).astype(jnp.bfloat16)
    f_mat = (r_idx[:, None] % NP == jnp.arange(N, dtype=jnp.int32)[None, :]
             ).astype(jnp.float32)

    out = pl.pallas_call(
        _tok_kernel,
        grid=(B // BT,),
        in_specs=[
            pl.BlockSpec((BT, N, D_IN), lambda i: (i, 0, 0)),
            pl.BlockSpec((BT, N), lambda i: (i, 0)),
            pl.BlockSpec((R, BT), lambda i: (0, 0)),
            pl.BlockSpec((R, N), lambda i: (0, 0)),
            pl.BlockSpec((D_IN, HIDDEN), lambda i: (0, 0)),
            pl.BlockSpec((D_IN, HIDDEN), lambda i: (0, 0)),
            pl.BlockSpec((1, HIDDEN), lambda i: (0, 0)),
            pl.BlockSpec((1, HIDDEN), lambda i: (0, 0)),
        ],
        out_specs=pl.BlockSpec((BT, N, HIDDEN), lambda i: (i, 0, 0)),
        out_shape=jax.ShapeDtypeStruct((B, N, HIDDEN), jnp.float32),
    )(x_feats, mf, e_mat, f_mat, W1, W2, b1r, b2r)
    return out
